# baseline (speedup 1.0000x reference)
import jax
import jax.numpy as jnp
from jax import lax
from jax.experimental import pallas as pl
from jax.experimental.pallas import tpu as pltpu
from jax.experimental.pallas import tpu_sc as plsc

N_USERS = 50000
N_NODES = 100000
DIM = 32
N_EDGES = 1600000
BATCH = 4096

NC = 2
NS = 16
EDGES_PER_SC = N_EDGES // NC
EDGES_PER_TILE = EDGES_PER_SC // NS
CH = 80
N_CHUNKS = EDGES_PER_TILE // CH
HALF = N_NODES // NC
ROWS_PER_TILE = HALF // NS
WB = 25
N_WB = ROWS_PER_TILE // WB
WBF = 3128
K = 5
NBLK = N_CHUNKS // K
BPT = BATCH // (NC * NS)

_mesh = plsc.VectorSubcoreMesh(
    core_axis_name="c", subcore_axis_name="s", num_cores=NC, num_subcores=NS
)


def _fire_block(cur, rows2, cols2, vals2, colb, rowb, valb, msgb, sem, c0):
    pltpu.sync_copy(cols2.at[pl.ds(c0, K)], colb)
    pltpu.sync_copy(rows2.at[pl.ds(c0, K)], rowb)
    pltpu.sync_copy(vals2.at[pl.ds(c0, K)], valb)
    for k in range(K):
        pltpu.async_copy(cur.at[colb.at[k]], msgb.at[k], sem)


def _drain_block(cur, acc, colb, rowb, valb, msgb, sem, rbase):
    for k in range(K):
        pltpu.make_async_copy(cur.at[colb.at[k]], msgb.at[k], sem).wait()
    z16i = jnp.zeros((16,), jnp.int32)
    for k in range(K):
        @pl.loop(0, CH)
        def _scale(e):
            v = plsc.load_gather(valb.at[k], [z16i + e])
            msgb[k, e, pl.ds(0, 16)] = msgb[k, e, pl.ds(0, 16)] * v
            msgb[k, e, pl.ds(16, 16)] = msgb[k, e, pl.ds(16, 16)] * v

        @pl.loop(0, CH // 16)
        def _rebase(h):
            rowb[k, pl.ds(h * 16, 16)] = rowb[k, pl.ds(h * 16, 16)] - rbase

        pltpu.sync_copy(msgb.at[k], acc.at[rowb.at[k]], add=True)


def _spmm_body(cur, rows2, cols2, vals2, out, acc,
               colA, rowA, valA, msgA, semA,
               colB, rowB, valB, msgB, semB, cpb):
    cid = lax.axis_index("c")
    sid = lax.axis_index("s")
    rbase = cid * HALF
    cbase = cid * (EDGES_PER_SC // CH) + sid * N_CHUNKS

    z16 = jnp.zeros((16,), jnp.float32)

    @pl.loop(0, WB)
    def _zero(r):
        cpb[r, pl.ds(0, 16)] = z16
        cpb[r, pl.ds(16, 16)] = z16

    @pl.loop(0, N_WB)
    def _zcopy(kk):
        pltpu.sync_copy(cpb, acc.at[pl.ds(sid * ROWS_PER_TILE + kk * WB, WB)])

    plsc.subcore_barrier()

    _fire_block(cur, rows2, cols2, vals2, colA, rowA, valA, msgA, semA,
                cbase)

    @pl.loop(0, (NBLK - 1) // 2)
    def _blk(t):
        b0 = 2 * t
        _fire_block(cur, rows2, cols2, vals2, colB, rowB, valB, msgB, semB,
                    cbase + (b0 + 1) * K)
        _drain_block(cur, acc, colA, rowA, valA, msgA, semA, rbase)

        @pl.when(t < (NBLK - 1) // 2 - 1)
        def _more():
            _fire_block(cur, rows2, cols2, vals2, colA, rowA, valA, msgA,
                        semA, cbase + (b0 + 2) * K)

        _drain_block(cur, acc, colB, rowB, valB, msgB, semB, rbase)

    _fire_block(cur, rows2, cols2, vals2, colA, rowA, valA, msgA, semA,
                cbase + (NBLK - 1) * K)
    _drain_block(cur, acc, colA, rowA, valA, msgA, semA, rbase)

    plsc.subcore_barrier()

    @pl.when(sid < NS - 1)
    def _wb_full():
        r0 = pl.multiple_of(sid * WBF, 8)
        pltpu.sync_copy(acc.at[pl.ds(r0, WBF)], out.at[pl.ds(rbase + r0, WBF)])

    @pl.when(sid == NS - 1)
    def _wb_tail():
        r0 = (NS - 1) * WBF
        pltpu.sync_copy(acc.at[pl.ds(r0, HALF - r0)],
                        out.at[pl.ds(rbase + r0, HALF - r0)])


_params = pltpu.CompilerParams(use_tc_tiling_on_sc=False,
                               needs_layout_passes=False)

_layer = pl.kernel(
    _spmm_body,
    out_type=jax.ShapeDtypeStruct((N_NODES, DIM), jnp.float32),
    mesh=_mesh,
    compiler_params=_params,
    scratch_types=(
        [pltpu.VMEM_SHARED((HALF, DIM), jnp.float32)]
        + 2 * [
            pltpu.VMEM((K, CH), jnp.int32),
            pltpu.VMEM((K, CH), jnp.int32),
            pltpu.VMEM((K, CH), jnp.float32),
            pltpu.VMEM((K, CH, DIM), jnp.float32),
            pltpu.SemaphoreType.DMA,
        ]
        + [pltpu.VMEM((WB, DIM), jnp.float32)]
    ),
)


def _score_body(e0, e1, e2, uin, pin, nin, out,
                uidx, pidx, nidx, g0u, g1u, g2u, g0p, g1p, g2p, g0n, g1n, g2n,
                ob, sem):
    cid = lax.axis_index("c")
    sid = lax.axis_index("s")
    wid = cid * NS + sid
    base = wid * BPT

    pltpu.sync_copy(uin.at[pl.ds(base, BPT)], uidx)
    pltpu.sync_copy(pin.at[pl.ds(base, BPT)], pidx)
    pltpu.sync_copy(nin.at[pl.ds(base, BPT)], nidx)

    @pl.loop(0, BPT // 16)
    def _adj(h):
        pidx[pl.ds(h * 16, 16)] = pidx[pl.ds(h * 16, 16)] + N_USERS
        nidx[pl.ds(h * 16, 16)] = nidx[pl.ds(h * 16, 16)] + N_USERS

    pltpu.async_copy(e0.at[uidx], g0u, sem).wait()
    pltpu.async_copy(e1.at[uidx], g1u, sem).wait()
    pltpu.async_copy(e2.at[uidx], g2u, sem).wait()
    pltpu.async_copy(e0.at[pidx], g0p, sem).wait()
    pltpu.async_copy(e1.at[pidx], g1p, sem).wait()
    pltpu.async_copy(e2.at[pidx], g2p, sem).wait()
    pltpu.async_copy(e0.at[nidx], g0n, sem).wait()
    pltpu.async_copy(e1.at[nidx], g1n, sem).wait()
    pltpu.async_copy(e2.at[nidx], g2n, sem).wait()

    @pl.loop(0, BPT // 16)
    def _dot(g):
        lane = lax.iota(jnp.int32, 16)
        res = jnp.zeros((16,), jnp.float32)
        for l in range(16):
            r = g * 16 + l
            u0 = g0u[r, pl.ds(0, 16)] + g1u[r, pl.ds(0, 16)] + g2u[r, pl.ds(0, 16)]
            u1 = g0u[r, pl.ds(16, 16)] + g1u[r, pl.ds(16, 16)] + g2u[r, pl.ds(16, 16)]
            d0 = (g0p[r, pl.ds(0, 16)] + g1p[r, pl.ds(0, 16)] + g2p[r, pl.ds(0, 16)]
                  - g0n[r, pl.ds(0, 16)] - g1n[r, pl.ds(0, 16)] - g2n[r, pl.ds(0, 16)])
            d1 = (g0p[r, pl.ds(16, 16)] + g1p[r, pl.ds(16, 16)] + g2p[r, pl.ds(16, 16)]
                  - g0n[r, pl.ds(16, 16)] - g1n[r, pl.ds(16, 16)] - g2n[r, pl.ds(16, 16)])
            s = jnp.sum(u0 * d0 + u1 * d1) * (1.0 / 9.0)
            res = jnp.where(lane == l, s, res)
        ob[pl.ds(g * 16, 16)] = res

    pltpu.sync_copy(ob, out.at[pl.ds(base, BPT)])


_scores = pl.kernel(
    _score_body,
    out_type=jax.ShapeDtypeStruct((BATCH,), jnp.float32),
    mesh=_mesh,
    compiler_params=_params,
    scratch_types=(
        [pltpu.VMEM((BPT,), jnp.int32) for _ in range(3)]
        + [pltpu.VMEM((BPT, DIM), jnp.float32) for _ in range(9)]
        + [pltpu.VMEM((BPT,), jnp.float32), pltpu.SemaphoreType.DMA]
    ),
)


def kernel(user_emb, item_emb, adj_row, adj_col, adj_val,
           user_input, pos_item_input, neg_item_input):
    e0 = jnp.concatenate([user_emb, item_emb], axis=0)
    rows2 = adj_row.reshape(-1, CH)
    cols2 = adj_col.reshape(-1, CH)
    vals2 = adj_val.reshape(-1, CH)
    e1 = _layer(e0, rows2, cols2, vals2)
    e2 = _layer(e1, rows2, cols2, vals2)
    s = _scores(e0, e1, e2,
                user_input.astype(jnp.int32),
                pos_item_input.astype(jnp.int32),
                neg_item_input.astype(jnp.int32))
    return s.reshape(BATCH, 1)


# speedup vs baseline: 2.2035x; 2.2035x over previous
import jax
import jax.numpy as jnp
from jax import lax
from jax.experimental import pallas as pl
from jax.experimental.pallas import tpu as pltpu
from jax.experimental.pallas import tpu_sc as plsc

N_USERS = 50000
N_NODES = 100000
DIM = 32
N_EDGES = 1600000
BATCH = 4096

NC = 2
NS = 16
EDGES_PER_SC = N_EDGES // NC
EDGES_PER_TILE = EDGES_PER_SC // NS
CH = 80
N_CHUNKS = EDGES_PER_TILE // CH
HALF = N_NODES // NC
ROWS_PER_TILE = HALF // NS
WB = 25
N_WB = ROWS_PER_TILE // WB
WBF = 3128
K = 5
NBLK = N_CHUNKS // K
BPT = BATCH // (NC * NS)

_mesh = plsc.VectorSubcoreMesh(
    core_axis_name="c", subcore_axis_name="s", num_cores=NC, num_subcores=NS
)


NBUF = 5
PF = 3
NB5 = N_CHUNKS // NBUF


def _fire_idx(rows2, cols2, vals2, colb, rowb, valb, isem, c0):
    pltpu.async_copy(cols2.at[c0], colb, isem)
    pltpu.async_copy(rows2.at[c0], rowb, isem)
    pltpu.async_copy(vals2.at[c0], valb, isem)


def _wait_idx(rows2, cols2, vals2, colb, rowb, valb, isem, c0):
    pltpu.make_async_copy(cols2.at[c0], colb, isem).wait()
    pltpu.make_async_copy(rows2.at[c0], rowb, isem).wait()
    pltpu.make_async_copy(vals2.at[c0], valb, isem).wait()


def _drain(cur, acc, colb, rowb, valb, msgb, gsem, ssem, rbase):
    pltpu.make_async_copy(cur.at[colb], msgb, gsem).wait()

    @pl.loop(0, CH // 16)
    def _scale(h):
        vv = valb[pl.ds(h * 16, 16)]
        for l in range(16):
            e = h * 16 + l
            v = vv[l]
            msgb[e, pl.ds(0, 16)] = msgb[e, pl.ds(0, 16)] * v
            msgb[e, pl.ds(16, 16)] = msgb[e, pl.ds(16, 16)] * v

    @pl.loop(0, CH // 16)
    def _rebase(h):
        rowb[pl.ds(h * 16, 16)] = rowb[pl.ds(h * 16, 16)] - rbase

    pltpu.sync_copy(msgb, acc.at[rowb], add=True)


def _spmm_body(cur, rows2, cols2, vals2, out, acc, *scr):
    sets = [scr[b * 7:(b + 1) * 7] for b in range(NBUF)]
    cpb = scr[-1]
    cid = lax.axis_index("c")
    sid = lax.axis_index("s")
    rbase = cid * HALF
    cbase = cid * (EDGES_PER_SC // CH) + sid * N_CHUNKS

    z16 = jnp.zeros((16,), jnp.float32)

    @pl.loop(0, WB)
    def _zero(r):
        cpb[r, pl.ds(0, 16)] = z16
        cpb[r, pl.ds(16, 16)] = z16

    @pl.loop(0, N_WB)
    def _zcopy(kk):
        pltpu.sync_copy(cpb, acc.at[pl.ds(sid * ROWS_PER_TILE + kk * WB, WB)])

    plsc.subcore_barrier()

    for b in range(PF):
        pltpu.sync_copy(cols2.at[cbase + b], sets[b][0])
        pltpu.sync_copy(rows2.at[cbase + b], sets[b][1])
        pltpu.sync_copy(vals2.at[cbase + b], sets[b][2])
    _fire_idx(rows2, cols2, vals2, *sets[3][:3], sets[3][4], cbase + PF)
    for b in range(PF):
        pltpu.async_copy(cur.at[sets[b][0]], sets[b][3], sets[b][5])

    @pl.loop(0, NB5)
    def _blk(t):
        blk0 = t * NBUF
        for b in range(NBUF):
            s4 = (b + 4) % NBUF
            s3 = (b + 3) % NBUF
            st4 = sets[s4]
            st3 = sets[s3]
            stb = sets[b]

            def _pf4(b=b, st4=st4):
                _fire_idx(rows2, cols2, vals2, *st4[:3], st4[4],
                          cbase + blk0 + b + 4)

            if b == 0:
                _pf4()
            else:
                @pl.when(t < NB5 - 1)
                def _c4():
                    _pf4()

            def _pf3(st3=st3, b=b):
                _wait_idx(rows2, cols2, vals2, *st3[:3], st3[4],
                          cbase + blk0 + b + PF)
                pltpu.async_copy(cur.at[st3[0]], st3[3], st3[5])

            if b < 2:
                _pf3()
            else:
                @pl.when(t < NB5 - 1)
                def _c3():
                    _pf3()

            _drain(cur, acc, stb[0], stb[1], stb[2], stb[3], stb[5],
                   stb[6], rbase)

    plsc.subcore_barrier()

    @pl.when(sid < NS - 1)
    def _wb_full():
        r0 = pl.multiple_of(sid * WBF, 8)
        pltpu.sync_copy(acc.at[pl.ds(r0, WBF)], out.at[pl.ds(rbase + r0, WBF)])

    @pl.when(sid == NS - 1)
    def _wb_tail():
        r0 = (NS - 1) * WBF
        pltpu.sync_copy(acc.at[pl.ds(r0, HALF - r0)],
                        out.at[pl.ds(rbase + r0, HALF - r0)])


_params = pltpu.CompilerParams(use_tc_tiling_on_sc=False,
                               needs_layout_passes=False)

_layer = pl.kernel(
    _spmm_body,
    out_type=jax.ShapeDtypeStruct((N_NODES, DIM), jnp.float32),
    mesh=_mesh,
    compiler_params=_params,
    scratch_types=(
        [pltpu.VMEM_SHARED((HALF, DIM), jnp.float32)]
        + NBUF * [
            pltpu.VMEM((CH,), jnp.int32),
            pltpu.VMEM((CH,), jnp.int32),
            pltpu.VMEM((CH,), jnp.float32),
            pltpu.VMEM((CH, DIM), jnp.float32),
            pltpu.SemaphoreType.DMA,
            pltpu.SemaphoreType.DMA,
            pltpu.SemaphoreType.DMA,
        ]
        + [pltpu.VMEM((WB, DIM), jnp.float32)]
    ),
)


def _score_body(e0, e1, e2, uin, pin, nin, out,
                uidx, pidx, nidx, g0u, g1u, g2u, g0p, g1p, g2p, g0n, g1n, g2n,
                ob, sem):
    cid = lax.axis_index("c")
    sid = lax.axis_index("s")
    wid = cid * NS + sid
    base = wid * BPT

    pltpu.sync_copy(uin.at[pl.ds(base, BPT)], uidx)
    pltpu.sync_copy(pin.at[pl.ds(base, BPT)], pidx)
    pltpu.sync_copy(nin.at[pl.ds(base, BPT)], nidx)

    @pl.loop(0, BPT // 16)
    def _adj(h):
        pidx[pl.ds(h * 16, 16)] = pidx[pl.ds(h * 16, 16)] + N_USERS
        nidx[pl.ds(h * 16, 16)] = nidx[pl.ds(h * 16, 16)] + N_USERS

    pltpu.async_copy(e0.at[uidx], g0u, sem).wait()
    pltpu.async_copy(e1.at[uidx], g1u, sem).wait()
    pltpu.async_copy(e2.at[uidx], g2u, sem).wait()
    pltpu.async_copy(e0.at[pidx], g0p, sem).wait()
    pltpu.async_copy(e1.at[pidx], g1p, sem).wait()
    pltpu.async_copy(e2.at[pidx], g2p, sem).wait()
    pltpu.async_copy(e0.at[nidx], g0n, sem).wait()
    pltpu.async_copy(e1.at[nidx], g1n, sem).wait()
    pltpu.async_copy(e2.at[nidx], g2n, sem).wait()

    @pl.loop(0, BPT // 16)
    def _dot(g):
        lane = lax.iota(jnp.int32, 16)
        res = jnp.zeros((16,), jnp.float32)
        for l in range(16):
            r = g * 16 + l
            u0 = g0u[r, pl.ds(0, 16)] + g1u[r, pl.ds(0, 16)] + g2u[r, pl.ds(0, 16)]
            u1 = g0u[r, pl.ds(16, 16)] + g1u[r, pl.ds(16, 16)] + g2u[r, pl.ds(16, 16)]
            d0 = (g0p[r, pl.ds(0, 16)] + g1p[r, pl.ds(0, 16)] + g2p[r, pl.ds(0, 16)]
                  - g0n[r, pl.ds(0, 16)] - g1n[r, pl.ds(0, 16)] - g2n[r, pl.ds(0, 16)])
            d1 = (g0p[r, pl.ds(16, 16)] + g1p[r, pl.ds(16, 16)] + g2p[r, pl.ds(16, 16)]
                  - g0n[r, pl.ds(16, 16)] - g1n[r, pl.ds(16, 16)] - g2n[r, pl.ds(16, 16)])
            s = jnp.sum(u0 * d0 + u1 * d1) * (1.0 / 9.0)
            res = jnp.where(lane == l, s, res)
        ob[pl.ds(g * 16, 16)] = res

    pltpu.sync_copy(ob, out.at[pl.ds(base, BPT)])


_scores = pl.kernel(
    _score_body,
    out_type=jax.ShapeDtypeStruct((BATCH,), jnp.float32),
    mesh=_mesh,
    compiler_params=_params,
    scratch_types=(
        [pltpu.VMEM((BPT,), jnp.int32) for _ in range(3)]
        + [pltpu.VMEM((BPT, DIM), jnp.float32) for _ in range(9)]
        + [pltpu.VMEM((BPT,), jnp.float32), pltpu.SemaphoreType.DMA]
    ),
)


def kernel(user_emb, item_emb, adj_row, adj_col, adj_val,
           user_input, pos_item_input, neg_item_input):
    e0 = jnp.concatenate([user_emb, item_emb], axis=0)
    rows2 = adj_row.reshape(-1, CH)
    cols2 = adj_col.reshape(-1, CH)
    vals2 = adj_val.reshape(-1, CH)
    e1 = _layer(e0, rows2, cols2, vals2)
    e2 = _layer(e1, rows2, cols2, vals2)
    s = _scores(e0, e1, e2,
                user_input.astype(jnp.int32),
                pos_item_input.astype(jnp.int32),
                neg_item_input.astype(jnp.int32))
    return s.reshape(BATCH, 1)
